# Initial kernel scaffold; baseline (speedup 1.0000x reference)
import jax
import jax.numpy as jnp
from jax.experimental import pallas as pl


def kernel(X):
    raise NotImplementedError("write your pallas kernel here")


# speedup vs baseline: 2570.3420x; 2570.3420x over previous
import functools

import jax
import jax.numpy as jnp
import numpy as np
from jax import lax
from jax.experimental import pallas as pl
from jax.experimental.pallas import tpu as pltpu
from jax.experimental.pallas import tpu_sc as plsc

BINS = 50
LO, HI = -1.0, 1.0
EPS = 1e-6
W = 128
R = 2048
NBP = 64
BR = 8
BPAD = 144
PCOLS = NBP * NBP
GROUP = 16

_NC, _NS = 2, 16
ROWS_PER_WORKER = R // (_NC * _NS)
NGROUPS = ROWS_PER_WORKER // GROUP


def _sc_body(x_hbm, bext_hbm, pzero_hbm, bins_hbm, p_hbm,
             x_v, bins_v, p_v, bext_v, mnmx_v):
    wid = lax.axis_index("s") * _NC + lax.axis_index("c")
    base = wid * ROWS_PER_WORKER

    pltpu.sync_copy(bext_hbm, bext_v)
    pltpu.sync_copy(x_hbm.at[pl.ds(base, ROWS_PER_WORKER)], x_v)
    pltpu.sync_copy(pzero_hbm, p_v)

    lanes = lax.broadcasted_iota(jnp.int32, (16,), 0)
    tail_mask = lanes < 15

    def row_pass(r, i):
        chunks = [x_v[r, pl.ds(16 * j, 16)] for j in range(8)]
        mn, mx = chunks[0], chunks[0]
        for c in chunks[1:]:
            mn = jnp.minimum(mn, c)
            mx = jnp.maximum(mx, c)
        for k in (8, 4, 2, 1):
            mnmx_v[pl.ds(0, 16)] = mn
            mnmx_v[pl.ds(16, 16)] = mx
            perm = lanes ^ k
            mn = jnp.minimum(mn, plsc.load_gather(mnmx_v, [perm]))
            mx = jnp.maximum(mx, plsc.load_gather(mnmx_v, [perm + 16]))
        rmin = mn
        rmax = mx
        d = rmax - rmin + EPS
        for j in range(8):
            xs = ((chunks[j] - rmin) / d) * (HI - LO) + LO
            e = jnp.clip(((xs - LO) * (BINS / (HI - LO))).astype(jnp.int32),
                         0, BINS - 1)
            blo = plsc.load_gather(bext_v, [e])
            bhi = plsc.load_gather(bext_v, [e + 1])
            e = (e + (bhi < xs).astype(jnp.int32)
                 - (blo >= xs).astype(jnp.int32))
            bins_v[r, pl.ds(16 * j, 16)] = e
        rowvec = jnp.full((16,), i, dtype=jnp.int32)
        for j in range(8):
            cur = bins_v[r, pl.ds(16 * j, 16)]
            nxt = bins_v[r, pl.ds(16 * j + 1, 16)]
            idx = cur * NBP + nxt
            valid = None if j < 7 else tail_mask
            cnt, last = plsc.scan_count(idx, valid)
            plsc.addupdate_scatter(p_v, [rowvec, idx],
                                   cnt.astype(jnp.float32), mask=last)

    def row_zero(r, i):
        rowvec = jnp.full((16,), i, dtype=jnp.int32)
        zeros = jnp.zeros((16,), dtype=jnp.float32)
        for j in range(8):
            cur = bins_v[r, pl.ds(16 * j, 16)]
            nxt = bins_v[r, pl.ds(16 * j + 1, 16)]
            idx = cur * NBP + nxt
            valid = None if j < 7 else tail_mask
            plsc.store_scatter(p_v, [rowvec, idx], zeros, mask=valid)

    def group_body(g, carry):
        def fill(i, carry):
            row_pass(g * GROUP + i, i)
            return carry
        lax.fori_loop(0, GROUP, fill, 0)
        pltpu.sync_copy(p_v, p_hbm.at[pl.ds(base + g * GROUP, GROUP)])

        def zero(i, carry):
            row_zero(g * GROUP + i, i)
            return carry
        lax.fori_loop(0, GROUP, zero, 0)
        return carry

    lax.fori_loop(0, NGROUPS, group_body, 0)
    pltpu.sync_copy(bins_v, bins_hbm.at[pl.ds(base, ROWS_PER_WORKER)])


def _expand_block(bins_row_ref, p_ref, out_ref):
    row_iota = lax.broadcasted_iota(jnp.int32, (NBP, W), 0)
    for r in range(BR):
        brow = bins_row_ref[r : r + 1, :]
        gt = (row_iota == brow).astype(jnp.bfloat16)
        p = p_ref[r].astype(jnp.bfloat16)
        m = jax.lax.dot_general(
            gt, p, (((0,), (0,)), ((), ())),
            preferred_element_type=jnp.float32)
        mbf = m.astype(jnp.bfloat16)
        s = jnp.sum(m, axis=1, keepdims=True)
        recip = 1.0 / jnp.where(s == 0.0, 1.0, s)
        out = jax.lax.dot_general(
            mbf, gt, (((1,), (0,)), ((), ())),
            preferred_element_type=jnp.float32)
        out_ref[r, :, :] = out * recip


@jax.jit
def kernel(X):
    lead = X.shape[:-1]
    Xf = X.reshape(R, W)

    boundaries = jnp.linspace(LO, HI, BINS + 1, dtype=X.dtype)[1:-1]
    bext = jnp.concatenate([
        jnp.array([-1e30], dtype=jnp.float32),
        boundaries.astype(jnp.float32),
        jnp.full((NBP - BINS,), 1e30, dtype=jnp.float32),
    ])
    pzero = jnp.zeros((GROUP, PCOLS), dtype=jnp.float32)

    sc = pl.kernel(
        _sc_body,
        out_type=[
            jax.ShapeDtypeStruct((R, BPAD), jnp.int32),
            jax.ShapeDtypeStruct((R, PCOLS), jnp.float32),
        ],
        mesh=plsc.VectorSubcoreMesh(core_axis_name="c", subcore_axis_name="s"),
        compiler_params=pltpu.CompilerParams(needs_layout_passes=False),
        scratch_types=[
            pltpu.VMEM((ROWS_PER_WORKER, W), jnp.float32),
            pltpu.VMEM((ROWS_PER_WORKER, BPAD), jnp.int32),
            pltpu.VMEM((GROUP, PCOLS), jnp.float32),
            pltpu.VMEM((NBP,), jnp.float32),
            pltpu.VMEM((32,), jnp.float32),
        ],
    )
    bins_pad, p_flat = sc(Xf, bext, pzero)
    bins = bins_pad[:, :W]
    p3 = p_flat.reshape(R, NBP, NBP)

    out = pl.pallas_call(
        _expand_block,
        grid=(R // BR,),
        in_specs=[
            pl.BlockSpec((BR, W), lambda i: (i, 0)),
            pl.BlockSpec((BR, NBP, NBP), lambda i: (i, 0, 0)),
        ],
        out_specs=pl.BlockSpec((BR, W, W), lambda i: (i, 0, 0)),
        out_shape=jax.ShapeDtypeStruct((R, W, W), jnp.float32),
    )(bins, p3)
    return out.reshape(lead + (W, W))


# speedup vs baseline: 3741.2450x; 1.4555x over previous
import functools

import jax
import jax.numpy as jnp
import numpy as np
from jax import lax
from jax.experimental import pallas as pl
from jax.experimental.pallas import tpu as pltpu
from jax.experimental.pallas import tpu_sc as plsc

BINS = 50
LO, HI = -1.0, 1.0
EPS = 1e-6
W = 128
R = 2048
NBP = 64
BR = 32
BPAD = 144
PCOLS = NBP * NBP
GROUP = 16

_NC, _NS = 2, 16
ROWS_PER_WORKER = R // (_NC * _NS)
NGROUPS = ROWS_PER_WORKER // GROUP


def _sc_body(x_hbm, bext_hbm, pzero_hbm, bins_hbm, p_hbm,
             x_v, bins_v, p_v, bext_v, mnmx_v):
    wid = lax.axis_index("s") * _NC + lax.axis_index("c")
    base = wid * ROWS_PER_WORKER

    pltpu.sync_copy(bext_hbm, bext_v)
    pltpu.sync_copy(x_hbm.at[pl.ds(base, ROWS_PER_WORKER)], x_v)
    pltpu.sync_copy(pzero_hbm, p_v)

    lanes = lax.broadcasted_iota(jnp.int32, (16,), 0)
    tail_mask = lanes < 15

    def row_pass(r, i):
        chunks = [x_v[r, pl.ds(16 * j, 16)] for j in range(8)]
        mn, mx = chunks[0], chunks[0]
        for c in chunks[1:]:
            mn = jnp.minimum(mn, c)
            mx = jnp.maximum(mx, c)
        for k in (8, 4, 2, 1):
            mnmx_v[pl.ds(0, 16)] = mn
            mnmx_v[pl.ds(16, 16)] = mx
            perm = lanes ^ k
            mn = jnp.minimum(mn, plsc.load_gather(mnmx_v, [perm]))
            mx = jnp.maximum(mx, plsc.load_gather(mnmx_v, [perm + 16]))
        rmin = mn
        rmax = mx
        d = rmax - rmin + EPS
        for j in range(8):
            xs = ((chunks[j] - rmin) / d) * (HI - LO) + LO
            e = jnp.clip(((xs - LO) * (BINS / (HI - LO))).astype(jnp.int32),
                         0, BINS - 1)
            blo = plsc.load_gather(bext_v, [e])
            bhi = plsc.load_gather(bext_v, [e + 1])
            e = (e + (bhi < xs).astype(jnp.int32)
                 - (blo >= xs).astype(jnp.int32))
            bins_v[r, pl.ds(16 * j, 16)] = e
        rowvec = jnp.full((16,), i, dtype=jnp.int32)
        for j in range(8):
            cur = bins_v[r, pl.ds(16 * j, 16)]
            nxt = bins_v[r, pl.ds(16 * j + 1, 16)]
            idx = cur * NBP + nxt
            valid = None if j < 7 else tail_mask
            cnt, last = plsc.scan_count(idx, valid)
            plsc.addupdate_scatter(p_v, [rowvec, idx],
                                   cnt.astype(jnp.float32), mask=last)

    def row_zero(r, i):
        rowvec = jnp.full((16,), i, dtype=jnp.int32)
        zeros = jnp.zeros((16,), dtype=jnp.float32)
        for j in range(8):
            cur = bins_v[r, pl.ds(16 * j, 16)]
            nxt = bins_v[r, pl.ds(16 * j + 1, 16)]
            idx = cur * NBP + nxt
            valid = None if j < 7 else tail_mask
            plsc.store_scatter(p_v, [rowvec, idx], zeros, mask=valid)

    def group_body(g, carry):
        def fill(i, carry):
            row_pass(g * GROUP + i, i)
            return carry
        lax.fori_loop(0, GROUP, fill, 0)
        pltpu.sync_copy(p_v, p_hbm.at[pl.ds(base + g * GROUP, GROUP)])

        def zero(i, carry):
            row_zero(g * GROUP + i, i)
            return carry
        lax.fori_loop(0, GROUP, zero, 0)
        return carry

    lax.fori_loop(0, NGROUPS, group_body, 0)
    pltpu.sync_copy(bins_v, bins_hbm.at[pl.ds(base, ROWS_PER_WORKER)])


def _expand_block(bins_row_ref, p_ref, out_ref):
    row_iota = lax.broadcasted_iota(jnp.int32, (NBP, W), 0)
    for r in range(BR):
        brow = bins_row_ref[r : r + 1, :]
        gt = (row_iota == brow).astype(jnp.bfloat16)
        p = p_ref[r].astype(jnp.bfloat16)
        m = jax.lax.dot_general(
            gt, p, (((0,), (0,)), ((), ())),
            preferred_element_type=jnp.float32)
        mbf = m.astype(jnp.bfloat16)
        s = jnp.sum(m, axis=1, keepdims=True)
        recip = 1.0 / jnp.where(s == 0.0, 1.0, s)
        out = jax.lax.dot_general(
            mbf, gt, (((1,), (0,)), ((), ())),
            preferred_element_type=jnp.float32)
        out_ref[r, :, :] = out * recip


@jax.jit
def kernel(X):
    lead = X.shape[:-1]
    Xf = X.reshape(R, W)

    boundaries = jnp.linspace(LO, HI, BINS + 1, dtype=X.dtype)[1:-1]
    bext = jnp.concatenate([
        jnp.array([-1e30], dtype=jnp.float32),
        boundaries.astype(jnp.float32),
        jnp.full((NBP - BINS,), 1e30, dtype=jnp.float32),
    ])
    pzero = jnp.zeros((GROUP, PCOLS), dtype=jnp.float32)

    sc = pl.kernel(
        _sc_body,
        out_type=[
            jax.ShapeDtypeStruct((R, BPAD), jnp.int32),
            jax.ShapeDtypeStruct((R, PCOLS), jnp.float32),
        ],
        mesh=plsc.VectorSubcoreMesh(core_axis_name="c", subcore_axis_name="s"),
        compiler_params=pltpu.CompilerParams(needs_layout_passes=False),
        scratch_types=[
            pltpu.VMEM((ROWS_PER_WORKER, W), jnp.float32),
            pltpu.VMEM((ROWS_PER_WORKER, BPAD), jnp.int32),
            pltpu.VMEM((GROUP, PCOLS), jnp.float32),
            pltpu.VMEM((NBP,), jnp.float32),
            pltpu.VMEM((32,), jnp.float32),
        ],
    )
    bins_pad, p_flat = sc(Xf, bext, pzero)
    bins = bins_pad[:, :W]
    p3 = p_flat.reshape(R, NBP, NBP)

    out = pl.pallas_call(
        _expand_block,
        grid=(R // BR,),
        in_specs=[
            pl.BlockSpec((BR, W), lambda i: (i, 0)),
            pl.BlockSpec((BR, NBP, NBP), lambda i: (i, 0, 0)),
        ],
        out_specs=pl.BlockSpec((BR, W, W), lambda i: (i, 0, 0)),
        out_shape=jax.ShapeDtypeStruct((R, W, W), jnp.float32),
    )(bins, p3)
    return out.reshape(lead + (W, W))
